# baseline (speedup 1.0000x reference)
import math

import numpy as np
import jax
import jax.numpy as jnp
from jax.experimental import pallas as pl

_DIM = 256
_HEADS = 8
_L = 15
_NFEAT = 143
_B = 128
_HD = _DIM // _HEADS
_TOK = _B * _L
_G = 8
_R = _G * _L
_NG = _B // _G
_PREC = jax.lax.Precision.HIGHEST
_PDEF = jax.lax.Precision.DEFAULT
_SCALE = 1.0 / math.sqrt(_HD)

_IDX_HEX = [
    '5242a91dba56363367a7e3153ed8b33a0371102544cad490528539153645350632e43702b4353c6953ce6b2d355d1251a60874893e5c0162984ae329d9b0522dca820bbd6260b782d964c9765e98bc5887a8911a231688495bc7bc2678b26898b5108445409592b42727841aed94bb16d',
    '797609602657568396e2d2708783378bb15ccb97495465b101e656d8cad0774743326730700aba7289cba744ba824e6e3da33eb9c725d786696c9c1da4d07bac2e5c2ee97a9d3ed734b97d36c81c7e9aa6d4d6a1a037cbc1db59e64e6de257695909b350869606842787711b0c5c36dd5',
    '140b555bec52c36dbdb653ed503de94d3ead8a3680a6d7c28a77ad9861a8a0e95de8978209ba920cab0ab6c9884c11c364e54ecb4246186b2878e500801c169927a487645aab6b4028365ac3d93c25c77c81a167162d394dc627529e1841be37c7cda80a385e6a9d1093162d092515a25',
    '4712bed886ad86463d9d21e3c488d6ed4e45bdc55bea03368522c8a7815d967badb9dbe28e9ec9eaa3b8c95e4190025ac63a196359211895695444040241c337e2c26334aad75d76a4c16b40903bcde3912d39d1280a5de4833c38dcd25139ea2d5e470cc55a545549d3a8b6a312b4ae0',
    '81884bd85b0c95e18c46a565cb46aea06b0756976730c289bc09b9007dbbe24ab85311712642eedce770bee097844cc9ac3e83d0451b03c512bddba93e14361a8cba3005b60892ae35451a3ce1608ab75603bbd3d70297023a6b7793e46021e0ce5cb7d462409e38630ba786244220a20',
    'd2e05cd1311188c6c26b55d5a3a5ab33b6ba50567cc1e7ccc7505b96820495938495cbbcc96d4650531d6aa011a93d8e8b757c8005a11b10c9607ad1d066188cb27ce745bd3036e7ab9ba4aaae5a748c761231bb2a22de99619a669e9e58ce85977b8c6e9c8559bb7904200681d614564',
    '3697db04aa40da6b6765142556ecad4b5633c1ba7359cdb21a3dc28b8c0756e7278432767e94be5b618592591773309724bd145060e0066c4a39725ce7d2de37cd0c2d5b6d02b545b36db25514ec6d5ee757e438145c2778b6e41a1ed4ea4b85cbc148cc7cddad60a71d34a6cb831913a',
    '2a0d0032845427c7c4493348006b13c4c85d596b843397e11c5046cd0ec6a28a6c8961e19833795e78b67e6bc431344e168589c43d14465348c882de04668714eb7846783eca5604b7c31bc46d992c5075b19390d3b1a32b8e4e07bd53870432d5297a4b81818949e356741a48b037d35',
    '1737e40db60378b81b4405db190124b474dabd516172cda897e5844e90cb929c6d622db036baaadb3d50b0a953aac5786abb90ed586e296823cba90407ab8927ec982080711b9b1c7a33eb96ec68a89889d7c36dd65d0d034ce336511ca34156ebeb8087dabe74cd077a6906690976066',
    '4ebc6c5d5d17a0d453ae9293a8d9ab615b8e42a638051891cd1d3ce657c5bab3a5ec37b9741e072396072165374ba6583a3c7414597bec220e92024c5561d3872ed9e9606b302ab618127d1c743e38088755cb95e46dd55ab5b107ea29b883cc8b69305221921561db61e813402b07e25',
    '63c91a4b2705a374180627875b4063e2c63d5b6831a902ea9578239729a4150973c5c3a88c9b85b158ea238e66b1972ea74919ace4795e297bc4ae5a860bd4e7cddda7818102ab35b791b0840072750403042177d87b584d80b1dcbcd547c046e4a9d8131bc0b65e217ead36abe40bebb',
    'cbd302c4ecb4e9ae95480755e5754e08d58335c5b2bd7eb36917b7edc4a153cd5c151cdc4da67175787a4079b00e10551766d74cb7e65d49a56c2971ee52cdadd596abee2434463c9b36b0d970501b874e8ddbcc79dad13133278c57dec8a54495555420d3431a156ebe76bc95672cd53',
]


def _build_consts():
    eid = np.repeat(np.arange(_G), _L)
    pos = np.tile(np.arange(_L), _G)
    inb = (eid[:, None] == eid[None, :])
    emask = (np.arange(_DIM)[None, :] // _HD == eid[:, None]).astype(np.float32)
    negbd = np.where(inb, 0.0, -np.inf).astype(np.float32)
    upt = (inb & (pos[:, None] < pos[None, :])).astype(np.float32)
    ndin = (inb & ~np.eye(_R, dtype=bool)).astype(np.float32)
    posmt = np.where(inb, pos[:, None], -1).astype(np.int32)
    sels, cnts = [], []
    for s_hex in _IDX_HEX:
        idx = np.array([int(ch, 16) for ch in s_hex],
                       np.int32).reshape(_L, _L)
        sel15 = np.full((_L, _L), -np.inf, np.float32)
        cnt15 = np.zeros((_L, _L), np.float32)
        for l in range(_L):
            for s in range(_L):
                sel15[l, idx[l, s]] = 0.0
                cnt15[l, idx[l, s]] += 1.0
        selbd = np.where(inb, sel15[pos[:, None], pos[None, :]],
                         -np.inf).astype(np.float32)
        cntbd = np.where(inb, cnt15[pos[:, None], pos[None, :]],
                         0.0).astype(np.float32)
        sels.append(selbd)
        cnts.append(cntbd)
    sels = np.stack(sels)
    cnts = np.stack(cnts)
    selsT = np.swapaxes(sels, 1, 2).copy()
    cntsT = np.swapaxes(cnts, 1, 2).copy()
    return (emask, negbd, upt, ndin, posmt, sels, cnts, selsT, cntsT)


(_EMASK, _NEGBD, _UPT, _NDIN, _POSMT,
 _SELBD, _CNTBD, _SELBDT, _CNTBDT) = _build_consts()


def _lin(x2, W, b):
    y = jax.lax.dot_general(x2, W, (((1,), (1,)), ((), ())),
                            precision=_PDEF)
    return y + b[None, :]


def _ln(x2, g, b):
    m = jnp.mean(x2, axis=-1, keepdims=True)
    v = jnp.mean((x2 - m) ** 2, axis=-1, keepdims=True)
    return (x2 - m) / jnp.sqrt(v + 1e-5) * g[None, :] + b[None, :]


def _gelu(x):
    return x * (jax.lax.erf(x / np.float32(np.sqrt(2))) + 1) / 2


def _attn_core(q2, k2, v2, emask, negbd, upt, ndin, posmt,
               selbd, cntbd, selbdT, cntbdT):
    n = selbd.shape[0]
    outs = []
    for h in range(_HEADS):
        sl = slice(h * _HD, (h + 1) * _HD)
        qh = q2[:, sl]
        kh = k2[:, sl]
        vh = v2[:, sl]
        bq = jnp.tile(qh, (1, _G)) * emask
        bk = jnp.tile(kh, (1, _G)) * emask
        S = jax.lax.dot_general(bq, bk, (((1,), (1,)), ((), ())),
                                precision=_PREC)
        P = S * _SCALE + negbd
        mx = jnp.max(P, axis=-1, keepdims=True)
        e = jnp.exp(P - mx)
        A = e / jnp.sum(e, axis=-1, keepdims=True)
        O = jax.lax.dot_general(A, vh, (((1,), (0,)), ((), ())),
                                precision=_PDEF)
        Pacc = None
        for c in range(n):
            Mc = (jnp.max(S + selbd[c], axis=1, keepdims=True)
                  - jnp.sum(S * cntbd[c], axis=1, keepdims=True)
                  * (1.0 / _L))
            Mr = jnp.transpose(Mc)
            T = jnp.where(((Mc > Mr) | ((Mc == Mr) & (upt > 0)))
                          & (ndin > 0), 1, 0)
            rank = jnp.sum(T, axis=0, keepdims=True)
            Pm = jnp.where(rank == posmt, 1.0, 0.0)
            Pacc = Pm if Pacc is None else Pacc + Pm
        if n > 1:
            Pacc = Pacc * (1.0 / n)
        oh = jax.lax.dot_general(Pacc, O, (((1,), (0,)), ((), ())),
                                 precision=_PDEF)
        outs.append(oh)
    return jnp.concatenate(outs, axis=-1)


def _shift(x3, o):
    if o == 0:
        return x3
    B, L, C = x3.shape
    z = jnp.zeros((B, abs(o), C), x3.dtype)
    if o > 0:
        return jnp.concatenate([x3[:, o:, :], z], axis=1)
    return jnp.concatenate([z, x3[:, :L + o, :]], axis=1)


def _conv1d(x2, Wf, b, K, dilation):
    I = x2.shape[1]
    x3 = x2.reshape(_G, _L, I)
    ke = (K - 1) * dilation + 1
    pad_left = (ke - 1) // 2
    y = None
    for t in range(K):
        off = t * dilation - pad_left
        if off >= _L or off <= -_L:
            continue
        xs = _shift(x3, off).reshape(_R, I)
        Wt = Wf[:, t * I:(t + 1) * I]
        term = jax.lax.dot_general(xs, Wt, (((1,), (1,)), ((), ())),
                                   precision=_PDEF)
        y = term if y is None else y + term
    return y + b[None, :]


def _wflat(cp):
    W = cp['W']
    O, I, K = W.shape
    return W.transpose(0, 2, 1).reshape(O, K * I)


def _k_embed(x_ref, t0W, t0b, tg, tb, t1W, t1b, feW, feb,
             m0W, m0b, m1W, m1b, m2W, m2b, m3W, m3b,
             em, nb, up, nd, pt, sel, cnt, selT, cntT, o_ref):
    x = x_ref[...]
    t = x[:, :15]
    f = x[:, 15:]
    te = _lin(t, t0W[...], t0b[...])
    te = _ln(te, tg[...], tb[...])
    te = _gelu(te)
    te = _lin(te, t1W[...], t1b[...])
    fe = _lin(f, feW[...], feb[...])
    h = jnp.concatenate([te, fe], axis=-1)
    a = _attn_core(h, h, h, em[...], nb[...], up[...], nd[...], pt[...],
                   sel[...], cnt[...], selT[...], cntT[...])
    h = h + a
    acc = None
    for i, (W, b) in enumerate(((m0W, m0b), (m1W, m1b), (m2W, m2b),
                                (m3W, m3b))):
        br = _gelu(_conv1d(h, W[...], b[...], 2 ** i, 2 ** i))
        acc = br if acc is None else acc + br
    o_ref[...] = h + acc * 0.25


def _k_enc(h_ref, qW, qb, kW, kb, vW, vb, oW, ob,
           c1W, c1b, c2W, c2b, m1W, m1b, m2W, m2b,
           g1, b1, g2, b2, g3, b3,
           em, nb, up, nd, pt, sel, cnt, selT, cntT, o_ref):
    h = h_ref[...]
    q = _lin(h, qW[...], qb[...])
    k = _lin(h, kW[...], kb[...])
    v = _lin(h, vW[...], vb[...])
    a = _attn_core(q, k, v, em[...], nb[...], up[...], nd[...], pt[...],
                   sel[...], cnt[...], selT[...], cntT[...])
    h = _ln(h + _lin(a, oW[...], ob[...]), g1[...], b1[...])
    c = _gelu(_conv1d(h, c1W[...], c1b[...], 5, 1))
    c = _conv1d(c, c2W[...], c2b[...], 5, 1)
    h = _ln(h + c, g2[...], b2[...])
    m = _lin(_gelu(_lin(h, m1W[...], m1b[...])), m2W[...], m2b[...])
    o_ref[...] = _ln(h + m, g3[...], b3[...])


def _k_dec(z_ref, mem_ref,
           sqW, sqb, skW, skb, svW, svb, soW, sob,
           cqW, cqb, ckW, ckb, cvW, cvb, coW, cob,
           c1W, c1b, c2W, c2b, m1W, m1b, m2W, m2b,
           g1, b1, g2, b2, g3, b3, g4, b4,
           em, nb, up, nd, pt, ssel, scnt, sselT, scntT,
           csel, ccnt, cselT, ccntT, o_ref):
    z = z_ref[...]
    mem = mem_ref[...]
    q = _lin(z, sqW[...], sqb[...])
    k = _lin(z, skW[...], skb[...])
    v = _lin(z, svW[...], svb[...])
    a = _attn_core(q, k, v, em[...], nb[...], up[...], nd[...], pt[...],
                   ssel[...], scnt[...], sselT[...], scntT[...])
    z = _ln(z + _lin(a, soW[...], sob[...]), g1[...], b1[...])
    q = _lin(z, cqW[...], cqb[...])
    k = _lin(mem, ckW[...], ckb[...])
    v = _lin(mem, cvW[...], cvb[...])
    a = _attn_core(q, k, v, em[...], nb[...], up[...], nd[...], pt[...],
                   csel[...], ccnt[...], cselT[...], ccntT[...])
    z = _ln(z + _lin(a, coW[...], cob[...]), g2[...], b2[...])
    c = _gelu(_conv1d(z, c1W[...], c1b[...], 5, 1))
    c = _conv1d(c, c2W[...], c2b[...], 5, 1)
    z = _ln(z + c, g3[...], b3[...])
    m = _lin(_gelu(_lin(z, m1W[...], m1b[...])), m2W[...], m2b[...])
    o_ref[...] = _ln(z + m, g4[...], b4[...])


def _k_head(z_ref, tW, tb, fW, fb, o_ref):
    z = z_ref[...]
    t_out = _lin(z[:, :_DIM // 4], tW[...], tb[...])
    f_out = _lin(z[:, _DIM // 4:], fW[...], fb[...])
    o_ref[...] = jnp.concatenate([t_out, f_out], axis=-1)


def _call(body, out_cols, act, *weights):
    weights = [jnp.asarray(w) for w in weights]
    in_specs = [pl.BlockSpec((_R, act.shape[1]), lambda i: (i, 0))]
    for w in weights:
        nd = w.ndim
        in_specs.append(
            pl.BlockSpec(w.shape, (lambda i, _n=nd: (0,) * _n)))
    return pl.pallas_call(
        body,
        grid=(_NG,),
        in_specs=in_specs,
        out_specs=pl.BlockSpec((_R, out_cols), lambda i: (i, 0)),
        out_shape=jax.ShapeDtypeStruct((_TOK, out_cols), jnp.float32),
    )(act, *weights)


def _call2(body, out_cols, act, act2, *weights):
    weights = [jnp.asarray(w) for w in weights]
    in_specs = [pl.BlockSpec((_R, act.shape[1]), lambda i: (i, 0)),
                pl.BlockSpec((_R, act2.shape[1]), lambda i: (i, 0))]
    for w in weights:
        nd = w.ndim
        in_specs.append(
            pl.BlockSpec(w.shape, (lambda i, _n=nd: (0,) * _n)))
    return pl.pallas_call(
        body,
        grid=(_NG,),
        in_specs=in_specs,
        out_specs=pl.BlockSpec((_R, out_cols), lambda i: (i, 0)),
        out_shape=jax.ShapeDtypeStruct((_TOK, out_cols), jnp.float32),
    )(act, act2, *weights)


def kernel(x, params):
    p = params
    xf = x.reshape(_TOK, _NFEAT)
    cm = (_EMASK, _NEGBD, _UPT, _NDIN, _POSMT)
    h = _call(
        _k_embed, _DIM, xf,
        p['t0']['W'], p['t0']['b'], p['tln']['g'], p['tln']['b'],
        p['t1']['W'], p['t1']['b'], p['femb']['W'], p['femb']['b'],
        _wflat(p['mstcn'][0]), p['mstcn'][0]['b'],
        _wflat(p['mstcn'][1]), p['mstcn'][1]['b'],
        _wflat(p['mstcn'][2]), p['mstcn'][2]['b'],
        _wflat(p['mstcn'][3]), p['mstcn'][3]['b'],
        *cm, _SELBD[0:3], _CNTBD[0:3], _SELBDT[0:3], _CNTBDT[0:3])
    c = 3
    for lp in p['enc']:
        h = _call(
            _k_enc, _DIM, h,
            lp['attn']['q']['W'], lp['attn']['q']['b'],
            lp['attn']['k']['W'], lp['attn']['k']['b'],
            lp['attn']['v']['W'], lp['attn']['v']['b'],
            lp['attn']['o']['W'], lp['attn']['o']['b'],
            _wflat(lp['c1']), lp['c1']['b'], _wflat(lp['c2']), lp['c2']['b'],
            lp['m1']['W'], lp['m1']['b'], lp['m2']['W'], lp['m2']['b'],
            lp['n1']['g'], lp['n1']['b'], lp['n2']['g'], lp['n2']['b'],
            lp['n3']['g'], lp['n3']['b'],
            *cm, _SELBD[c:c + 1], _CNTBD[c:c + 1],
            _SELBDT[c:c + 1], _CNTBDT[c:c + 1])
        c += 1
    mem = h
    z = h
    for lp in p['dec']:
        z = _call2(
            _k_dec, _DIM, z, mem,
            lp['sattn']['q']['W'], lp['sattn']['q']['b'],
            lp['sattn']['k']['W'], lp['sattn']['k']['b'],
            lp['sattn']['v']['W'], lp['sattn']['v']['b'],
            lp['sattn']['o']['W'], lp['sattn']['o']['b'],
            lp['cattn']['q']['W'], lp['cattn']['q']['b'],
            lp['cattn']['k']['W'], lp['cattn']['k']['b'],
            lp['cattn']['v']['W'], lp['cattn']['v']['b'],
            lp['cattn']['o']['W'], lp['cattn']['o']['b'],
            _wflat(lp['c1']), lp['c1']['b'], _wflat(lp['c2']), lp['c2']['b'],
            lp['m1']['W'], lp['m1']['b'], lp['m2']['W'], lp['m2']['b'],
            lp['n1']['g'], lp['n1']['b'], lp['n2']['g'], lp['n2']['b'],
            lp['n3']['g'], lp['n3']['b'], lp['n4']['g'], lp['n4']['b'],
            *cm, _SELBD[c:c + 1], _CNTBD[c:c + 1],
            _SELBDT[c:c + 1], _CNTBDT[c:c + 1],
            _SELBD[c + 1:c + 2], _CNTBD[c + 1:c + 2],
            _SELBDT[c + 1:c + 2], _CNTBDT[c + 1:c + 2])
        c += 2
    out = _call(_k_head, _NFEAT, z,
                p['tout']['W'], p['tout']['b'],
                p['fout']['W'], p['fout']['b'])
    return out.reshape(_B, _L, _NFEAT)


# speedup vs baseline: 1.2324x; 1.2324x over previous
import math

import numpy as np
import jax
import jax.numpy as jnp
from jax.experimental import pallas as pl

_DIM = 256
_HEADS = 8
_L = 15
_NFEAT = 143
_B = 128
_HD = _DIM // _HEADS
_TOK = _B * _L
_G = 8
_R = _G * _L
_NG = _B // _G
_PREC = jax.lax.Precision.HIGHEST
_PDEF = jax.lax.Precision.DEFAULT
_SCALE = 1.0 / math.sqrt(_HD)

_IDX_HEX = [
    '5242a91dba56363367a7e3153ed8b33a0371102544cad490528539153645350632e43702b4353c6953ce6b2d355d1251a60874893e5c0162984ae329d9b0522dca820bbd6260b782d964c9765e98bc5887a8911a231688495bc7bc2678b26898b5108445409592b42727841aed94bb16d',
    '797609602657568396e2d2708783378bb15ccb97495465b101e656d8cad0774743326730700aba7289cba744ba824e6e3da33eb9c725d786696c9c1da4d07bac2e5c2ee97a9d3ed734b97d36c81c7e9aa6d4d6a1a037cbc1db59e64e6de257695909b350869606842787711b0c5c36dd5',
    '140b555bec52c36dbdb653ed503de94d3ead8a3680a6d7c28a77ad9861a8a0e95de8978209ba920cab0ab6c9884c11c364e54ecb4246186b2878e500801c169927a487645aab6b4028365ac3d93c25c77c81a167162d394dc627529e1841be37c7cda80a385e6a9d1093162d092515a25',
    '4712bed886ad86463d9d21e3c488d6ed4e45bdc55bea03368522c8a7815d967badb9dbe28e9ec9eaa3b8c95e4190025ac63a196359211895695444040241c337e2c26334aad75d76a4c16b40903bcde3912d39d1280a5de4833c38dcd25139ea2d5e470cc55a545549d3a8b6a312b4ae0',
    '81884bd85b0c95e18c46a565cb46aea06b0756976730c289bc09b9007dbbe24ab85311712642eedce770bee097844cc9ac3e83d0451b03c512bddba93e14361a8cba3005b60892ae35451a3ce1608ab75603bbd3d70297023a6b7793e46021e0ce5cb7d462409e38630ba786244220a20',
    'd2e05cd1311188c6c26b55d5a3a5ab33b6ba50567cc1e7ccc7505b96820495938495cbbcc96d4650531d6aa011a93d8e8b757c8005a11b10c9607ad1d066188cb27ce745bd3036e7ab9ba4aaae5a748c761231bb2a22de99619a669e9e58ce85977b8c6e9c8559bb7904200681d614564',
    '3697db04aa40da6b6765142556ecad4b5633c1ba7359cdb21a3dc28b8c0756e7278432767e94be5b618592591773309724bd145060e0066c4a39725ce7d2de37cd0c2d5b6d02b545b36db25514ec6d5ee757e438145c2778b6e41a1ed4ea4b85cbc148cc7cddad60a71d34a6cb831913a',
    '2a0d0032845427c7c4493348006b13c4c85d596b843397e11c5046cd0ec6a28a6c8961e19833795e78b67e6bc431344e168589c43d14465348c882de04668714eb7846783eca5604b7c31bc46d992c5075b19390d3b1a32b8e4e07bd53870432d5297a4b81818949e356741a48b037d35',
    '1737e40db60378b81b4405db190124b474dabd516172cda897e5844e90cb929c6d622db036baaadb3d50b0a953aac5786abb90ed586e296823cba90407ab8927ec982080711b9b1c7a33eb96ec68a89889d7c36dd65d0d034ce336511ca34156ebeb8087dabe74cd077a6906690976066',
    '4ebc6c5d5d17a0d453ae9293a8d9ab615b8e42a638051891cd1d3ce657c5bab3a5ec37b9741e072396072165374ba6583a3c7414597bec220e92024c5561d3872ed9e9606b302ab618127d1c743e38088755cb95e46dd55ab5b107ea29b883cc8b69305221921561db61e813402b07e25',
    '63c91a4b2705a374180627875b4063e2c63d5b6831a902ea9578239729a4150973c5c3a88c9b85b158ea238e66b1972ea74919ace4795e297bc4ae5a860bd4e7cddda7818102ab35b791b0840072750403042177d87b584d80b1dcbcd547c046e4a9d8131bc0b65e217ead36abe40bebb',
    'cbd302c4ecb4e9ae95480755e5754e08d58335c5b2bd7eb36917b7edc4a153cd5c151cdc4da67175787a4079b00e10551766d74cb7e65d49a56c2971ee52cdadd596abee2434463c9b36b0d970501b874e8ddbcc79dad13133278c57dec8a54495555420d3431a156ebe76bc95672cd53',
]


def _build_consts():
    eid = np.repeat(np.arange(_G), _L)
    pos = np.tile(np.arange(_L), _G)
    inb = (eid[:, None] == eid[None, :])
    emask = (np.arange(_DIM)[None, :] // _HD == eid[:, None]).astype(np.float32)
    negbd = np.where(inb, 0.0, -np.inf).astype(np.float32)
    upt = (inb & (pos[:, None] < pos[None, :])).astype(np.float32)
    ndin = (inb & ~np.eye(_R, dtype=bool)).astype(np.float32)
    posmt = np.where(inb, pos[:, None], -1).astype(np.int32)
    sels, cnts = [], []
    for s_hex in _IDX_HEX:
        idx = np.array([int(ch, 16) for ch in s_hex],
                       np.int32).reshape(_L, _L)
        sel15 = np.full((_L, _L), -np.inf, np.float32)
        cnt15 = np.zeros((_L, _L), np.float32)
        for l in range(_L):
            for s in range(_L):
                sel15[l, idx[l, s]] = 0.0
                cnt15[l, idx[l, s]] += 1.0
        selbd = np.where(inb, sel15[pos[:, None], pos[None, :]],
                         -np.inf).astype(np.float32)
        cntbd = np.where(inb, cnt15[pos[:, None], pos[None, :]],
                         0.0).astype(np.float32)
        sels.append(selbd)
        cnts.append(cntbd)
    sels = np.stack(sels)
    cnts = np.stack(cnts)
    selsT = np.swapaxes(sels, 1, 2).copy()
    cntsT = np.swapaxes(cnts, 1, 2).copy()
    return (emask, negbd, upt, ndin, posmt, sels, cnts, selsT, cntsT)


(_EMASK, _NEGBD, _UPT, _NDIN, _POSMT,
 _SELBD, _CNTBD, _SELBDT, _CNTBDT) = _build_consts()


def _lin(x2, W, b):
    y = jax.lax.dot_general(x2, W, (((1,), (1,)), ((), ())),
                            precision=_PDEF)
    return y + b[None, :]


def _ln(x2, g, b):
    m = jnp.mean(x2, axis=-1, keepdims=True)
    v = jnp.mean((x2 - m) ** 2, axis=-1, keepdims=True)
    return (x2 - m) / jnp.sqrt(v + 1e-5) * g[None, :] + b[None, :]


def _gelu(x):
    return x * (jax.lax.erf(x / np.float32(np.sqrt(2))) + 1) / 2


def _attn_core(q2, k2, v2, emask, negbd, upt, ndin, posmt,
               selbd, cntbd, selbdT, cntbdT):
    n = selbd.shape[0]
    outs = []
    for h in range(_HEADS):
        sl = slice(h * _HD, (h + 1) * _HD)
        qh = q2[:, sl]
        kh = k2[:, sl]
        vh = v2[:, sl]
        bq = jnp.tile(qh, (1, _G)) * emask
        bk = jnp.tile(kh, (1, _G)) * emask
        def _dd(a, b):
            return jax.lax.dot_general(a, b, (((1,), (1,)), ((), ())),
                                       precision=_PDEF)
        bqh = bq.astype(jnp.bfloat16).astype(jnp.float32)
        bql = bq - bqh
        bkh = bk.astype(jnp.bfloat16).astype(jnp.float32)
        bkl = bk - bkh
        S = (_dd(bqh, bkh) + _dd(bqh, bkl)) + _dd(bql, bkh)
        St = (_dd(bkh, bqh) + _dd(bkl, bqh)) + _dd(bkh, bql)
        P = S * _SCALE + negbd
        mx = jnp.max(P, axis=-1, keepdims=True)
        e = jnp.exp(P - mx)
        A = e / jnp.sum(e, axis=-1, keepdims=True)
        O = jax.lax.dot_general(A, vh, (((1,), (0,)), ((), ())),
                                precision=_PDEF)
        Pacc = None
        for c in range(n):
            Mc = (jnp.max(S + selbd[c], axis=1, keepdims=True)
                  - jnp.sum(S * cntbd[c], axis=1, keepdims=True)
                  * (1.0 / _L))
            Mr = (jnp.max(St + selbdT[c], axis=0, keepdims=True)
                  - jnp.sum(St * cntbdT[c], axis=0, keepdims=True)
                  * (1.0 / _L))
            T = jnp.where(((Mc > Mr) | ((Mc == Mr) & (upt > 0)))
                          & (ndin > 0), 1, 0)
            rank = jnp.sum(T, axis=0, keepdims=True)
            Pm = jnp.where(rank == posmt, 1.0, 0.0)
            Pacc = Pm if Pacc is None else Pacc + Pm
        if n > 1:
            Pacc = Pacc * (1.0 / n)
        oh = jax.lax.dot_general(Pacc, O, (((1,), (0,)), ((), ())),
                                 precision=_PDEF)
        outs.append(oh)
    return jnp.concatenate(outs, axis=-1)


def _shift(x3, o):
    if o == 0:
        return x3
    B, L, C = x3.shape
    z = jnp.zeros((B, abs(o), C), x3.dtype)
    if o > 0:
        return jnp.concatenate([x3[:, o:, :], z], axis=1)
    return jnp.concatenate([z, x3[:, :L + o, :]], axis=1)


def _conv1d(x2, Wf, b, K, dilation):
    I = x2.shape[1]
    x3 = x2.reshape(_G, _L, I)
    ke = (K - 1) * dilation + 1
    pad_left = (ke - 1) // 2
    y = None
    for t in range(K):
        off = t * dilation - pad_left
        if off >= _L or off <= -_L:
            continue
        xs = _shift(x3, off).reshape(_R, I)
        Wt = Wf[:, t * I:(t + 1) * I]
        term = jax.lax.dot_general(xs, Wt, (((1,), (1,)), ((), ())),
                                   precision=_PDEF)
        y = term if y is None else y + term
    return y + b[None, :]


def _wflat(cp):
    W = cp['W']
    O, I, K = W.shape
    return W.transpose(0, 2, 1).reshape(O, K * I)


def _k_embed(x_ref, t0W, t0b, tg, tb, t1W, t1b, feW, feb,
             m0W, m0b, m1W, m1b, m2W, m2b, m3W, m3b,
             em, nb, up, nd, pt, sel, cnt, selT, cntT, o_ref):
    x = x_ref[...]
    t = x[:, :15]
    f = x[:, 15:]
    te = _lin(t, t0W[...], t0b[...])
    te = _ln(te, tg[...], tb[...])
    te = _gelu(te)
    te = _lin(te, t1W[...], t1b[...])
    fe = _lin(f, feW[...], feb[...])
    h = jnp.concatenate([te, fe], axis=-1)
    a = _attn_core(h, h, h, em[...], nb[...], up[...], nd[...], pt[...],
                   sel[...], cnt[...], selT[...], cntT[...])
    h = h + a
    acc = None
    for i, (W, b) in enumerate(((m0W, m0b), (m1W, m1b), (m2W, m2b),
                                (m3W, m3b))):
        br = _gelu(_conv1d(h, W[...], b[...], 2 ** i, 2 ** i))
        acc = br if acc is None else acc + br
    o_ref[...] = h + acc * 0.25


def _k_enc(h_ref, qW, qb, kW, kb, vW, vb, oW, ob,
           c1W, c1b, c2W, c2b, m1W, m1b, m2W, m2b,
           g1, b1, g2, b2, g3, b3,
           em, nb, up, nd, pt, sel, cnt, selT, cntT, o_ref):
    h = h_ref[...]
    q = _lin(h, qW[...], qb[...])
    k = _lin(h, kW[...], kb[...])
    v = _lin(h, vW[...], vb[...])
    a = _attn_core(q, k, v, em[...], nb[...], up[...], nd[...], pt[...],
                   sel[...], cnt[...], selT[...], cntT[...])
    h = _ln(h + _lin(a, oW[...], ob[...]), g1[...], b1[...])
    c = _gelu(_conv1d(h, c1W[...], c1b[...], 5, 1))
    c = _conv1d(c, c2W[...], c2b[...], 5, 1)
    h = _ln(h + c, g2[...], b2[...])
    m = _lin(_gelu(_lin(h, m1W[...], m1b[...])), m2W[...], m2b[...])
    o_ref[...] = _ln(h + m, g3[...], b3[...])


def _k_dec(z_ref, mem_ref,
           sqW, sqb, skW, skb, svW, svb, soW, sob,
           cqW, cqb, ckW, ckb, cvW, cvb, coW, cob,
           c1W, c1b, c2W, c2b, m1W, m1b, m2W, m2b,
           g1, b1, g2, b2, g3, b3, g4, b4,
           em, nb, up, nd, pt, ssel, scnt, sselT, scntT,
           csel, ccnt, cselT, ccntT, o_ref):
    z = z_ref[...]
    mem = mem_ref[...]
    q = _lin(z, sqW[...], sqb[...])
    k = _lin(z, skW[...], skb[...])
    v = _lin(z, svW[...], svb[...])
    a = _attn_core(q, k, v, em[...], nb[...], up[...], nd[...], pt[...],
                   ssel[...], scnt[...], sselT[...], scntT[...])
    z = _ln(z + _lin(a, soW[...], sob[...]), g1[...], b1[...])
    q = _lin(z, cqW[...], cqb[...])
    k = _lin(mem, ckW[...], ckb[...])
    v = _lin(mem, cvW[...], cvb[...])
    a = _attn_core(q, k, v, em[...], nb[...], up[...], nd[...], pt[...],
                   csel[...], ccnt[...], cselT[...], ccntT[...])
    z = _ln(z + _lin(a, coW[...], cob[...]), g2[...], b2[...])
    c = _gelu(_conv1d(z, c1W[...], c1b[...], 5, 1))
    c = _conv1d(c, c2W[...], c2b[...], 5, 1)
    z = _ln(z + c, g3[...], b3[...])
    m = _lin(_gelu(_lin(z, m1W[...], m1b[...])), m2W[...], m2b[...])
    o_ref[...] = _ln(z + m, g4[...], b4[...])


def _k_head(z_ref, tW, tb, fW, fb, o_ref):
    z = z_ref[...]
    t_out = _lin(z[:, :_DIM // 4], tW[...], tb[...])
    f_out = _lin(z[:, _DIM // 4:], fW[...], fb[...])
    o_ref[...] = jnp.concatenate([t_out, f_out], axis=-1)


def _call(body, out_cols, act, *weights):
    weights = [jnp.asarray(w) for w in weights]
    in_specs = [pl.BlockSpec((_R, act.shape[1]), lambda i: (i, 0))]
    for w in weights:
        nd = w.ndim
        in_specs.append(
            pl.BlockSpec(w.shape, (lambda i, _n=nd: (0,) * _n)))
    return pl.pallas_call(
        body,
        grid=(_NG,),
        in_specs=in_specs,
        out_specs=pl.BlockSpec((_R, out_cols), lambda i: (i, 0)),
        out_shape=jax.ShapeDtypeStruct((_TOK, out_cols), jnp.float32),
    )(act, *weights)


def _call2(body, out_cols, act, act2, *weights):
    weights = [jnp.asarray(w) for w in weights]
    in_specs = [pl.BlockSpec((_R, act.shape[1]), lambda i: (i, 0)),
                pl.BlockSpec((_R, act2.shape[1]), lambda i: (i, 0))]
    for w in weights:
        nd = w.ndim
        in_specs.append(
            pl.BlockSpec(w.shape, (lambda i, _n=nd: (0,) * _n)))
    return pl.pallas_call(
        body,
        grid=(_NG,),
        in_specs=in_specs,
        out_specs=pl.BlockSpec((_R, out_cols), lambda i: (i, 0)),
        out_shape=jax.ShapeDtypeStruct((_TOK, out_cols), jnp.float32),
    )(act, act2, *weights)


def kernel(x, params):
    p = params
    xf = x.reshape(_TOK, _NFEAT)
    cm = (_EMASK, _NEGBD, _UPT, _NDIN, _POSMT)
    h = _call(
        _k_embed, _DIM, xf,
        p['t0']['W'], p['t0']['b'], p['tln']['g'], p['tln']['b'],
        p['t1']['W'], p['t1']['b'], p['femb']['W'], p['femb']['b'],
        _wflat(p['mstcn'][0]), p['mstcn'][0]['b'],
        _wflat(p['mstcn'][1]), p['mstcn'][1]['b'],
        _wflat(p['mstcn'][2]), p['mstcn'][2]['b'],
        _wflat(p['mstcn'][3]), p['mstcn'][3]['b'],
        *cm, _SELBD[0:3], _CNTBD[0:3], _SELBDT[0:3], _CNTBDT[0:3])
    c = 3
    for lp in p['enc']:
        h = _call(
            _k_enc, _DIM, h,
            lp['attn']['q']['W'], lp['attn']['q']['b'],
            lp['attn']['k']['W'], lp['attn']['k']['b'],
            lp['attn']['v']['W'], lp['attn']['v']['b'],
            lp['attn']['o']['W'], lp['attn']['o']['b'],
            _wflat(lp['c1']), lp['c1']['b'], _wflat(lp['c2']), lp['c2']['b'],
            lp['m1']['W'], lp['m1']['b'], lp['m2']['W'], lp['m2']['b'],
            lp['n1']['g'], lp['n1']['b'], lp['n2']['g'], lp['n2']['b'],
            lp['n3']['g'], lp['n3']['b'],
            *cm, _SELBD[c:c + 1], _CNTBD[c:c + 1],
            _SELBDT[c:c + 1], _CNTBDT[c:c + 1])
        c += 1
    mem = h
    z = h
    for lp in p['dec']:
        z = _call2(
            _k_dec, _DIM, z, mem,
            lp['sattn']['q']['W'], lp['sattn']['q']['b'],
            lp['sattn']['k']['W'], lp['sattn']['k']['b'],
            lp['sattn']['v']['W'], lp['sattn']['v']['b'],
            lp['sattn']['o']['W'], lp['sattn']['o']['b'],
            lp['cattn']['q']['W'], lp['cattn']['q']['b'],
            lp['cattn']['k']['W'], lp['cattn']['k']['b'],
            lp['cattn']['v']['W'], lp['cattn']['v']['b'],
            lp['cattn']['o']['W'], lp['cattn']['o']['b'],
            _wflat(lp['c1']), lp['c1']['b'], _wflat(lp['c2']), lp['c2']['b'],
            lp['m1']['W'], lp['m1']['b'], lp['m2']['W'], lp['m2']['b'],
            lp['n1']['g'], lp['n1']['b'], lp['n2']['g'], lp['n2']['b'],
            lp['n3']['g'], lp['n3']['b'], lp['n4']['g'], lp['n4']['b'],
            *cm, _SELBD[c:c + 1], _CNTBD[c:c + 1],
            _SELBDT[c:c + 1], _CNTBDT[c:c + 1],
            _SELBD[c + 1:c + 2], _CNTBD[c + 1:c + 2],
            _SELBDT[c + 1:c + 2], _CNTBDT[c + 1:c + 2])
        c += 2
    out = _call(_k_head, _NFEAT, z,
                p['tout']['W'], p['tout']['b'],
                p['fout']['W'], p['fout']['b'])
    return out.reshape(_B, _L, _NFEAT)
